# Initial kernel scaffold; baseline (speedup 1.0000x reference)
import functools

import jax
import jax.numpy as jnp
from jax import lax
from jax.experimental import pallas as pl
from jax.experimental.pallas import tpu as pltpu
from jax.experimental.pallas import tpu_sc as plsc

EMB = 64
HASH = 64
N = 16384
L = 20
HID = 32
SAMPLE = 100000

NC, NS = 2, 16
NW = NC * NS
R_TOTAL = L * N
R_PER_W = R_TOTAL // NW
CHUNK = 128
N_CHUNK = R_PER_W // CHUNK


def _sc_gather_fn():
    mesh = plsc.VectorSubcoreMesh(core_axis_name="c", subcore_axis_name="s")

    @functools.partial(
        pl.kernel,
        out_type=jax.ShapeDtypeStruct((R_TOTAL, EMB), jnp.float32),
        mesh=mesh,
        scratch_types=[
            pltpu.VMEM((SAMPLE,), jnp.int32),
            pltpu.VMEM((CHUNK,), jnp.int32),
            pltpu.VMEM((CHUNK,), jnp.int32),
            pltpu.VMEM((CHUNK, EMB), jnp.float32),
            pltpu.SemaphoreType.DMA,
        ],
    )
    def k(spi_hbm, obf_hbm, emb_hbm, out_hbm, obf_v, idx_v, oidx_v, rows_v, sem):
        wid = lax.axis_index("s") * NC + lax.axis_index("c")
        base = pl.multiple_of(wid * R_PER_W, CHUNK)
        pltpu.sync_copy(obf_hbm, obf_v)

        def chunk_body(ci, _):
            r0 = pl.multiple_of(base + ci * CHUNK, CHUNK)
            pltpu.sync_copy(spi_hbm.at[pl.ds(r0, CHUNK)], idx_v)
            for j in range(CHUNK // 16):
                v = idx_v[pl.ds(j * 16, 16)]
                oidx_v[pl.ds(j * 16, 16)] = plsc.load_gather(obf_v, [v])
            pltpu.async_copy(emb_hbm.at[oidx_v], rows_v, sem).wait()
            pltpu.sync_copy(rows_v, out_hbm.at[pl.ds(r0, CHUNK)])
            return 0

        lax.fori_loop(0, N_CHUNK, chunk_body, 0)

    return k


_sc_gather = _sc_gather_fn()

BN = 512
GRID = N // BN


def _tc_body(emb_r, hash_r, mask_r, whT_r, wc1_r, wc2_r, bc_r, wf_r, bfin_r,
             wihf_r, whhf_r, bf_r, wihb_r, whhb_r, bb_r, a1_r, a2_r, v_r,
             out_r, xs_r, hf_r, hb_r):
    whT = whT_r[...]
    wc1 = wc1_r[...]
    wc2 = wc2_r[...]
    bc = bc_r[...]
    wf = wf_r[...]
    bfin = bfin_r[...]

    def lstm_pass(wih, whh, b, hout_r, t_order):
        h = jnp.zeros((BN, HID), jnp.float32)
        c = jnp.zeros((BN, HID), jnp.float32)
        for t in t_order:
            m = mask_r[:, t:t + 1]
            x = xs_r[t]
            g = jnp.dot(x, wih) + jnp.dot(h, whh) + b
            i_g = jax.nn.sigmoid(g[:, 0:HID])
            f_g = jax.nn.sigmoid(g[:, HID:2 * HID])
            g_g = jnp.tanh(g[:, 2 * HID:3 * HID])
            o_g = jax.nn.sigmoid(g[:, 3 * HID:4 * HID])
            c_new = f_g * c + i_g * g_g
            h_new = o_g * jnp.tanh(c_new)
            h = m * h_new + (1.0 - m) * h
            c = m * c_new + (1.0 - m) * c
            hout_r[t] = h

    for t in range(L):
        m = mask_r[:, t:t + 1]
        e = emb_r[t]
        hsh = hash_r[:, t, :]
        hp = jnp.dot(hsh, whT)
        comb = jnp.dot(e, wc1) + jnp.dot(hp, wc2) + bc
        x = jnp.maximum(comb, 0.0)
        x = jnp.dot(x, wf) + bfin
        xs_r[t] = jnp.maximum(x, 0.0) * m

    lstm_pass(wihf_r[...], whhf_r[...], bf_r[...], hf_r, range(L))
    lstm_pass(wihb_r[...], whhb_r[...], bb_r[...], hb_r, range(L - 1, -1, -1))

    a1 = a1_r[...]
    a2 = a2_r[...]
    v = v_r[...]
    cols = []
    for t in range(L):
        ht = jnp.tanh(jnp.dot(hf_r[t], a1) + jnp.dot(hb_r[t], a2))
        cols.append(jnp.sum(ht * v, axis=1, keepdims=True))
    s = jnp.concatenate(cols, axis=1)
    s = jnp.where(mask_r[...] > 0.5, s, -1e9)
    s = s - jnp.max(s, axis=1, keepdims=True)
    ex = jnp.exp(s)
    attn = ex / jnp.sum(ex, axis=1, keepdims=True)
    accf = jnp.zeros((BN, HID), jnp.float32)
    accb = jnp.zeros((BN, HID), jnp.float32)
    for t in range(L):
        a = attn[:, t:t + 1]
        accf = accf + a * hf_r[t]
        accb = accb + a * hb_r[t]
    out_r[...] = jnp.concatenate([accf, accb], axis=1)


def _tc_encode(emb3, hashings, maskf, *weights):
    def full(shape):
        return pl.BlockSpec(shape, lambda i: (0,) * len(shape))

    in_specs = [
        pl.BlockSpec((L, BN, EMB), lambda i: (0, i, 0)),
        pl.BlockSpec((BN, L, HASH), lambda i: (i, 0, 0)),
        pl.BlockSpec((BN, L), lambda i: (i, 0)),
    ] + [full(w.shape) for w in weights]
    return pl.pallas_call(
        _tc_body,
        grid=(GRID,),
        in_specs=in_specs,
        out_specs=pl.BlockSpec((BN, EMB), lambda i: (i, 0)),
        out_shape=jax.ShapeDtypeStruct((N, EMB), jnp.float32),
        scratch_shapes=[
            pltpu.VMEM((L, BN, EMB), jnp.float32),
            pltpu.VMEM((L, BN, HID), jnp.float32),
            pltpu.VMEM((L, BN, HID), jnp.float32),
        ],
        compiler_params=pltpu.CompilerParams(
            dimension_semantics=("parallel",),
        ),
    )(emb3, hashings, maskf, *weights)


def kernel(sub_parts_index, obf_sample, mask, lengths, hashings, emb_table,
           W_hash, W_comb, b_comb, W_final, b_final,
           W_ih_f, W_hh_f, b_f, W_ih_b, W_hh_b, b_b, W_att, v_att):
    spi_t = sub_parts_index.T.reshape(R_TOTAL).astype(jnp.int32)
    emb_flat = _sc_gather(spi_t, obf_sample.astype(jnp.int32), emb_table)
    emb3 = emb_flat.reshape(L, N, EMB)

    maskf = mask.astype(jnp.float32)
    wcT = W_comb.T
    waT = W_att.T
    weights = (
        W_hash.T,
        wcT[:EMB, :],
        wcT[EMB:, :],
        b_comb.reshape(1, -1),
        W_final.T,
        b_final.reshape(1, -1),
        W_ih_f.T, W_hh_f.T, b_f.reshape(1, -1),
        W_ih_b.T, W_hh_b.T, b_b.reshape(1, -1),
        waT[:HID, :],
        waT[HID:, :],
        v_att.reshape(1, -1),
    )
    return _tc_encode(emb3, hashings, maskf, *weights)


# speedup vs baseline: 3.6537x; 3.6537x over previous
import functools

import jax
import jax.numpy as jnp
from jax import lax
from jax.experimental import pallas as pl
from jax.experimental.pallas import tpu as pltpu
from jax.experimental.pallas import tpu_sc as plsc

EMB = 64
HASH = 64
N = 16384
L = 20
HID = 32
SAMPLE = 100000
SAMPLE_PAD = 100096

NC, NS = 2, 16
NW = NC * NS
R_TOTAL = L * N
R_PER_W = R_TOTAL // NW
CHUNK = 128
N_CHUNK = R_PER_W // CHUNK


@functools.cache
def _sc_gather_fn():
    mesh = plsc.VectorSubcoreMesh(
        core_axis_name="c", subcore_axis_name="s", num_cores=NC, num_subcores=NS)

    @functools.partial(
        pl.kernel,
        out_type=jax.ShapeDtypeStruct((R_TOTAL, EMB), jnp.float32),
        mesh=mesh,
        scratch_types=[
            pltpu.VMEM((CHUNK,), jnp.int32),
            pltpu.VMEM((CHUNK,), jnp.int32),
            pltpu.VMEM((CHUNK, EMB), jnp.float32),
            pltpu.SemaphoreType.DMA,
        ],
        compiler_params=pltpu.CompilerParams(use_tc_tiling_on_sc=False),
    )
    def k(spi_hbm, obf_hbm, emb_hbm, out_hbm, idx_v, oidx_v, rows_v, sem):
        wid = lax.axis_index("s") * NC + lax.axis_index("c")
        base = pl.multiple_of(wid * R_PER_W, CHUNK)

        def chunk_body(ci, _):
            r0 = pl.multiple_of(base + ci * CHUNK, CHUNK)
            pltpu.sync_copy(spi_hbm.at[pl.ds(r0, CHUNK)], idx_v)
            pltpu.async_copy(obf_hbm.at[idx_v], oidx_v, sem).wait()
            pltpu.async_copy(emb_hbm.at[oidx_v], rows_v, sem).wait()
            pltpu.sync_copy(rows_v, out_hbm.at[pl.ds(r0, CHUNK)])
            return 0

        lax.fori_loop(0, N_CHUNK, chunk_body, 0)

    return k

BN = 256
GRID = N // BN


def _tc_body(emb_r, hash_r, mask_r, whT_r, wc1_r, wc2_r, bc_r, wf_r, bfin_r,
             wihf_r, whhf_r, bf_r, wihb_r, whhb_r, bb_r, a1_r, a2_r, v_r,
             out_r, xs_r, hf_r, hb_r):
    whT = whT_r[...]
    wc1 = wc1_r[...]
    wc2 = wc2_r[...]
    bc = bc_r[...]
    wf = wf_r[...]
    bfin = bfin_r[...]

    def lstm_pass(wih, whh, b, hout_r, t_order):
        h = jnp.zeros((BN, HID), jnp.float32)
        c = jnp.zeros((BN, HID), jnp.float32)
        for t in t_order:
            m = mask_r[:, t:t + 1]
            x = xs_r[t]
            g = jnp.dot(x, wih) + jnp.dot(h, whh) + b
            i_g = jax.nn.sigmoid(g[:, 0:HID])
            f_g = jax.nn.sigmoid(g[:, HID:2 * HID])
            g_g = jnp.tanh(g[:, 2 * HID:3 * HID])
            o_g = jax.nn.sigmoid(g[:, 3 * HID:4 * HID])
            c_new = f_g * c + i_g * g_g
            h_new = o_g * jnp.tanh(c_new)
            h = m * h_new + (1.0 - m) * h
            c = m * c_new + (1.0 - m) * c
            hout_r[t] = h

    for t in range(L):
        m = mask_r[:, t:t + 1]
        e = emb_r[t]
        hsh = hash_r[:, t, :]
        hp = jnp.dot(hsh, whT)
        comb = jnp.dot(e, wc1) + jnp.dot(hp, wc2) + bc
        x = jnp.maximum(comb, 0.0)
        x = jnp.dot(x, wf) + bfin
        xs_r[t] = jnp.maximum(x, 0.0) * m

    lstm_pass(wihf_r[...], whhf_r[...], bf_r[...], hf_r, range(L))
    lstm_pass(wihb_r[...], whhb_r[...], bb_r[...], hb_r, range(L - 1, -1, -1))

    a1 = a1_r[...]
    a2 = a2_r[...]
    v = v_r[...]
    cols = []
    for t in range(L):
        ht = jnp.tanh(jnp.dot(hf_r[t], a1) + jnp.dot(hb_r[t], a2))
        cols.append(jnp.sum(ht * v, axis=1, keepdims=True))
    s = jnp.concatenate(cols, axis=1)
    s = jnp.where(mask_r[...] > 0.5, s, -1e9)
    s = s - jnp.max(s, axis=1, keepdims=True)
    ex = jnp.exp(s)
    attn = ex / jnp.sum(ex, axis=1, keepdims=True)
    accf = jnp.zeros((BN, HID), jnp.float32)
    accb = jnp.zeros((BN, HID), jnp.float32)
    for t in range(L):
        a = attn[:, t:t + 1]
        accf = accf + a * hf_r[t]
        accb = accb + a * hb_r[t]
    out_r[...] = jnp.concatenate([accf, accb], axis=1)


def _tc_encode(emb3, hashings, maskf, *weights):
    def full(shape):
        return pl.BlockSpec(shape, lambda i: (0,) * len(shape))

    in_specs = [
        pl.BlockSpec((L, BN, EMB), lambda i: (0, i, 0)),
        pl.BlockSpec((BN, L, HASH), lambda i: (i, 0, 0)),
        pl.BlockSpec((BN, L), lambda i: (i, 0)),
    ] + [full(w.shape) for w in weights]
    return pl.pallas_call(
        _tc_body,
        grid=(GRID,),
        in_specs=in_specs,
        out_specs=pl.BlockSpec((BN, EMB), lambda i: (i, 0)),
        out_shape=jax.ShapeDtypeStruct((N, EMB), jnp.float32),
        scratch_shapes=[
            pltpu.VMEM((L, BN, EMB), jnp.float32),
            pltpu.VMEM((L, BN, HID), jnp.float32),
            pltpu.VMEM((L, BN, HID), jnp.float32),
        ],
        compiler_params=pltpu.CompilerParams(
            dimension_semantics=("parallel",),
        ),
    )(emb3, hashings, maskf, *weights)


def kernel(sub_parts_index, obf_sample, mask, lengths, hashings, emb_table,
           W_hash, W_comb, b_comb, W_final, b_final,
           W_ih_f, W_hh_f, b_f, W_ih_b, W_hh_b, b_b, W_att, v_att):
    spi_t = sub_parts_index.T.reshape(R_TOTAL).astype(jnp.int32)
    obf_pad = jnp.pad(obf_sample.astype(jnp.int32), (0, SAMPLE_PAD - SAMPLE))
    emb_flat = _sc_gather_fn()(spi_t, obf_pad, emb_table)
    emb3 = emb_flat.reshape(L, N, EMB)

    maskf = mask.astype(jnp.float32)
    wcT = W_comb.T
    waT = W_att.T
    weights = (
        W_hash.T,
        wcT[:EMB, :],
        wcT[EMB:, :],
        b_comb.reshape(1, -1),
        W_final.T,
        b_final.reshape(1, -1),
        W_ih_f.T, W_hh_f.T, b_f.reshape(1, -1),
        W_ih_b.T, W_hh_b.T, b_b.reshape(1, -1),
        waT[:HID, :],
        waT[HID:, :],
        v_att.reshape(1, -1),
    )
    return _tc_encode(emb3, hashings, maskf, *weights)
